# baseline (speedup 1.0000x reference)
import functools

import jax
import jax.numpy as jnp
from jax import lax
from jax.experimental import pallas as pl
from jax.experimental.pallas import tpu as pltpu
from jax.experimental.pallas import tpu_sc as plsc

N_NODES = 10000
D_IN = 256
D_OUT = 256
N_EDGES = 160000

HALF = D_OUT // 2
N_TILES = 16
E_TILE = N_EDGES // N_TILES
K = 64
N_CHUNKS = 160
E_PAD = N_CHUNKS * K - E_TILE
ACC_ROWS = 10240
DUMMY_ROW = N_NODES + 64
ROWS_TILE = ACC_ROWS // N_TILES
OUT_TILE = 624


def _mm_body(x_ref, w_ref, o_ref):
    o_ref[...] = jnp.dot(x_ref[...], w_ref[...],
                         preferred_element_type=jnp.float32)


def _matmul_halves(x, W):
    BN = 1000
    return pl.pallas_call(
        _mm_body,
        grid=(2, N_NODES // BN),
        in_specs=[
            pl.BlockSpec((BN, D_IN), lambda c, i: (i, 0)),
            pl.BlockSpec((D_IN, HALF), lambda c, i: (0, c)),
        ],
        out_specs=pl.BlockSpec((BN, HALF), lambda c, i: (c * (N_NODES // BN) + i, 0)),
        out_shape=jax.ShapeDtypeStruct((2 * N_NODES, HALF), jnp.float32),
    )(x, W)


def _sc_aggregate(h2, idx2, zeros):
    mesh = plsc.VectorSubcoreMesh(core_axis_name="c", subcore_axis_name="s")

    @functools.partial(
        pl.kernel,
        mesh=mesh,
        out_type=jax.ShapeDtypeStruct((N_NODES, D_OUT), jnp.float32),
        scratch_types=[
            pltpu.VMEM((2, K), jnp.int32),
            pltpu.VMEM((2, K), jnp.int32),
            pltpu.VMEM((2, K), jnp.int32),
            pltpu.VMEM((2, K), jnp.int32),
            pltpu.VMEM((K, HALF), jnp.float32),
            pltpu.VMEM((K, HALF), jnp.float32),
            pltpu.VMEM((K, HALF), jnp.float32),
            pltpu.VMEM((K, HALF), jnp.float32),
            pltpu.VMEM_SHARED((ACC_ROWS, HALF), jnp.float32),
            pltpu.SemaphoreType.DMA,
            pltpu.SemaphoreType.DMA,
        ],
    )
    def agg(h2_hbm, idx2_hbm, zeros_hbm, out_hbm,
            ib0, ib1, ib2, ib3, db0, db1, db2, db3, acc, gsem, isem):
        c = lax.axis_index("c")
        s = lax.axis_index("s")
        ibufs = (ib0, ib1, ib2, ib3)
        dbufs = (db0, db1, db2, db3)

        pltpu.sync_copy(zeros_hbm.at[pl.ds(s * ROWS_TILE, ROWS_TILE)],
                        acc.at[pl.ds(s * ROWS_TILE, ROWS_TILE)])
        plsc.subcore_barrier()

        def fetch_idx(j, ib):
            pltpu.async_copy(idx2_hbm.at[c, s, j], ib, isem)

        def wait_idx(j, ib):
            pltpu.make_async_copy(idx2_hbm.at[c, s, j], ib, isem).wait()

        def start_gather(ib, db):
            pltpu.async_copy(h2_hbm.at[ib.at[0]], db, gsem)

        def wait_gather(ib, db):
            pltpu.make_async_copy(h2_hbm.at[ib.at[0]], db, gsem).wait()

        def scatter_add(ib, db):
            pltpu.sync_copy(db, acc.at[ib.at[1]], add=True)

        for j in range(3):
            fetch_idx(j, ibufs[j])
        fetch_idx(3, ib3)
        for j in range(3):
            wait_idx(j, ibufs[j])
            start_gather(ibufs[j], dbufs[j])

        def body(i, carry):
            j0 = 4 * i
            for p in range(4):
                j = j0 + p
                wait_idx(j + 3, ibufs[(p + 3) % 4])
                start_gather(ibufs[(p + 3) % 4], dbufs[(p + 3) % 4])
                wait_gather(ibufs[p], dbufs[p])
                scatter_add(ibufs[p], dbufs[p])
                fetch_idx(j + 4, ibufs[p])
            return carry

        lax.fori_loop(0, N_CHUNKS // 4 - 1, body, 0)

        wait_idx(N_CHUNKS - 1, ib3)
        start_gather(ib3, db3)
        wait_gather(ib0, db0)
        scatter_add(ib0, db0)

        wait_gather(ib1, db1)
        scatter_add(ib1, db1)
        wait_gather(ib2, db2)
        scatter_add(ib2, db2)
        wait_gather(ib3, db3)
        scatter_add(ib3, db3)

        plsc.subcore_barrier()

        col = pl.multiple_of(c * HALF, HALF)
        pltpu.sync_copy(acc.at[pl.ds(s * OUT_TILE, OUT_TILE)],
                        out_hbm.at[pl.ds(s * OUT_TILE, OUT_TILE), pl.ds(col, HALF)])

        @pl.when(s == N_TILES - 1)
        def _():
            base = N_TILES * OUT_TILE
            pltpu.sync_copy(acc.at[pl.ds(base, N_NODES - base)],
                            out_hbm.at[pl.ds(base, N_NODES - base), pl.ds(col, HALF)])

    return agg(h2, idx2, zeros)


def kernel(x, edge_index, W):
    src = edge_index[0].astype(jnp.int32)
    dst = edge_index[1].astype(jnp.int32)

    h2 = _matmul_halves(x, W)

    srcp = jnp.pad(src.reshape(N_TILES, E_TILE), ((0, 0), (0, E_PAD)),
                   constant_values=0).reshape(N_TILES, N_CHUNKS, K)
    dstp = jnp.pad(dst.reshape(N_TILES, E_TILE), ((0, 0), (0, E_PAD)),
                   constant_values=DUMMY_ROW).reshape(N_TILES, N_CHUNKS, K)
    idx2 = jnp.stack([
        jnp.stack([srcp, dstp], axis=2),
        jnp.stack([srcp + N_NODES, dstp], axis=2),
    ])
    zeros = jnp.zeros((ACC_ROWS, HALF), jnp.float32)

    return _sc_aggregate(h2, idx2, zeros)


# speedup vs baseline: 1.1518x; 1.1518x over previous
import functools

import jax
import jax.numpy as jnp
from jax import lax
from jax.experimental import pallas as pl
from jax.experimental.pallas import tpu as pltpu
from jax.experimental.pallas import tpu_sc as plsc

N_NODES = 10000
D_IN = 256
D_OUT = 256
N_EDGES = 160000

HALF = D_OUT // 2
N_TILES = 16
E_TILE = N_EDGES // N_TILES
K = 128
N_CHUNKS = 80
E_PAD = N_CHUNKS * K - E_TILE
ACC_ROWS = 10240
DUMMY_ROW = N_NODES + 64
ROWS_TILE = ACC_ROWS // N_TILES
OUT_TILE = 624


def _mm_body(x_ref, w_ref, o_ref):
    o_ref[...] = jnp.dot(x_ref[...], w_ref[...],
                         preferred_element_type=jnp.float32)


def _matmul_halves(x, W):
    BN = 1000
    return pl.pallas_call(
        _mm_body,
        grid=(2, N_NODES // BN),
        in_specs=[
            pl.BlockSpec((BN, D_IN), lambda c, i: (i, 0)),
            pl.BlockSpec((D_IN, HALF), lambda c, i: (0, c)),
        ],
        out_specs=pl.BlockSpec((BN, HALF), lambda c, i: (c * (N_NODES // BN) + i, 0)),
        out_shape=jax.ShapeDtypeStruct((2 * N_NODES, HALF), jnp.float32),
    )(x, W)


def _sc_aggregate(h2, idx2, zeros):
    mesh = plsc.VectorSubcoreMesh(core_axis_name="c", subcore_axis_name="s")

    @functools.partial(
        pl.kernel,
        mesh=mesh,
        out_type=jax.ShapeDtypeStruct((2, ACC_ROWS, HALF), jnp.float32),
        scratch_types=[
            pltpu.VMEM((2, K), jnp.int32),
            pltpu.VMEM((2, K), jnp.int32),
            pltpu.VMEM((2, K), jnp.int32),
            pltpu.VMEM((2, K), jnp.int32),
            pltpu.VMEM((K, HALF), jnp.float32),
            pltpu.VMEM((K, HALF), jnp.float32),
            pltpu.VMEM_SHARED((ACC_ROWS, HALF), jnp.float32),
            pltpu.SemaphoreType.DMA,
            pltpu.SemaphoreType.DMA,
        ],
    )
    def agg(h2_hbm, idx2_hbm, zeros_hbm, out_hbm,
            ib0, ib1, ib2, ib3, db0, db1, acc, gsem, isem):
        c = lax.axis_index("c")
        s = lax.axis_index("s")
        ibufs = (ib0, ib1, ib2, ib3)
        dbufs = (db0, db1)

        with jax.named_scope("acc_init"):
            pltpu.sync_copy(zeros_hbm.at[pl.ds(s * ROWS_TILE, ROWS_TILE)],
                            acc.at[pl.ds(s * ROWS_TILE, ROWS_TILE)])
            plsc.subcore_barrier()

        def fetch_idx(j, ib):
            pltpu.async_copy(idx2_hbm.at[c, s, j], ib, isem)

        def wait_idx(j, ib):
            pltpu.make_async_copy(idx2_hbm.at[c, s, j], ib, isem).wait()

        def start_gather(ib, db):
            pltpu.async_copy(h2_hbm.at[ib.at[0]], db, gsem)

        def wait_gather(ib, db):
            pltpu.make_async_copy(h2_hbm.at[ib.at[0]], db, gsem).wait()

        def scatter_add(ib, db):
            pltpu.sync_copy(db, acc.at[ib.at[1]], add=True)

        fetch_idx(0, ib0)
        fetch_idx(1, ib1)
        fetch_idx(2, ib2)
        wait_idx(0, ib0)
        start_gather(ib0, db0)

        def body(i, carry):
            j0 = 4 * i
            for p in range(4):
                j = j0 + p
                wait_idx(j + 1, ibufs[(p + 1) % 4])
                start_gather(ibufs[(p + 1) % 4], dbufs[(p + 1) % 2])
                wait_gather(ibufs[p], dbufs[p % 2])
                scatter_add(ibufs[p], dbufs[p % 2])
                fetch_idx(j + 3, ibufs[(p + 3) % 4])
            return carry

        lax.fori_loop(0, N_CHUNKS // 4 - 1, body, 0)

        wait_idx(N_CHUNKS - 3, ib1)
        start_gather(ib1, db1)
        wait_gather(ib0, db0)
        scatter_add(ib0, db0)
        fetch_idx(N_CHUNKS - 1, ib3)

        wait_idx(N_CHUNKS - 2, ib2)
        start_gather(ib2, db0)
        wait_gather(ib1, db1)
        scatter_add(ib1, db1)

        wait_idx(N_CHUNKS - 1, ib3)
        start_gather(ib3, db1)
        wait_gather(ib2, db0)
        scatter_add(ib2, db0)

        wait_gather(ib3, db1)
        scatter_add(ib3, db1)

        with jax.named_scope("copy_out"):
            plsc.subcore_barrier()
            pltpu.sync_copy(acc.at[pl.ds(s * ROWS_TILE, ROWS_TILE)],
                            out_hbm.at[c, pl.ds(s * ROWS_TILE, ROWS_TILE)])

    return agg(h2, idx2, zeros)


def kernel(x, edge_index, W):
    src = edge_index[0].astype(jnp.int32)
    dst = edge_index[1].astype(jnp.int32)

    h2 = _matmul_halves(x, W)

    srcp = jnp.pad(src.reshape(N_TILES, E_TILE), ((0, 0), (0, E_PAD)),
                   constant_values=0).reshape(N_TILES, N_CHUNKS, K)
    dstp = jnp.pad(dst.reshape(N_TILES, E_TILE), ((0, 0), (0, E_PAD)),
                   constant_values=DUMMY_ROW).reshape(N_TILES, N_CHUNKS, K)
    idx2 = jnp.stack([
        jnp.stack([srcp, dstp], axis=2),
        jnp.stack([srcp + N_NODES, dstp], axis=2),
    ])
    zeros = jnp.zeros((ACC_ROWS, HALF), jnp.float32)

    out2 = _sc_aggregate(h2, idx2, zeros)
    return jnp.concatenate([out2[0, :N_NODES], out2[1, :N_NODES]], axis=1)


# speedup vs baseline: 1.7024x; 1.4780x over previous
import functools

import jax
import jax.numpy as jnp
from jax import lax
from jax.experimental import pallas as pl
from jax.experimental.pallas import tpu as pltpu
from jax.experimental.pallas import tpu_sc as plsc

N_NODES = 10000
D_IN = 256
D_OUT = 256
N_EDGES = 160000

HALF = D_OUT // 2
N_TILES = 16
E_TILE = N_EDGES // N_TILES
K = 96
N_CHUNKS = 105
E_TILE_P = N_CHUNKS * K
E_PAD = E_TILE_P - E_TILE
ACC_ROWS = 10112
DUMMY_ROW = N_NODES + 16
ROWS_TILE = ACC_ROWS // N_TILES
OUT_TILE = 624


def _mm_body(x_ref, w_ref, o_ref):
    o_ref[...] = jnp.dot(x_ref[...], w_ref[...],
                         preferred_element_type=jnp.float32)


def _matmul_halves(x, W):
    BN = 1000
    return pl.pallas_call(
        _mm_body,
        grid=(2, N_NODES // BN),
        in_specs=[
            pl.BlockSpec((BN, D_IN), lambda c, i: (i, 0)),
            pl.BlockSpec((D_IN, HALF), lambda c, i: (0, c)),
        ],
        out_specs=pl.BlockSpec((BN, HALF), lambda c, i: (c * (N_NODES // BN) + i, 0)),
        out_shape=jax.ShapeDtypeStruct((2 * N_NODES, HALF), jnp.float32),
    )(x, W)


def _sc_aggregate(h2, src2, dst_r, zeros):
    mesh = plsc.VectorSubcoreMesh(core_axis_name="c", subcore_axis_name="s")

    @functools.partial(
        pl.kernel,
        mesh=mesh,
        out_type=jax.ShapeDtypeStruct((N_NODES, D_OUT), jnp.float32),
        scratch_types=[
            pltpu.VMEM((E_TILE_P,), jnp.int32),
            pltpu.VMEM((N_CHUNKS, K), jnp.int32),
            pltpu.VMEM((K, HALF), jnp.float32),
            pltpu.VMEM((K, HALF), jnp.float32),
            pltpu.VMEM_SHARED((ACC_ROWS, HALF), jnp.float32),
            pltpu.SemaphoreType.DMA,
        ],
    )
    def agg(h2_hbm, src2_hbm, dst_hbm, zeros_hbm, out_hbm,
            src_v, dst_v, db0, db1, acc, gsem):
        c = lax.axis_index("c")
        s = lax.axis_index("s")

        with jax.named_scope("stage"):
            pltpu.sync_copy(zeros_hbm.at[pl.ds(s * ROWS_TILE, ROWS_TILE)],
                            acc.at[pl.ds(s * ROWS_TILE, ROWS_TILE)])
            pltpu.sync_copy(
                src2_hbm.at[pl.ds((c * N_TILES + s) * E_TILE_P, E_TILE_P)],
                src_v)
            pltpu.sync_copy(dst_hbm.at[s], dst_v)
            plsc.subcore_barrier()

        def start_gather(j, db):
            pltpu.async_copy(h2_hbm.at[src_v.at[pl.ds(j * K, K)]], db, gsem)

        def wait_gather(j, db):
            pltpu.make_async_copy(h2_hbm.at[src_v.at[pl.ds(j * K, K)]], db,
                                  gsem).wait()

        def scatter_add(j, db):
            pltpu.sync_copy(db, acc.at[dst_v.at[j]], add=True)

        start_gather(0, db0)

        def body(i, carry):
            j = 2 * i
            start_gather(j + 1, db1)
            wait_gather(j, db0)
            scatter_add(j, db0)
            start_gather(j + 2, db0)
            wait_gather(j + 1, db1)
            scatter_add(j + 1, db1)
            return carry

        lax.fori_loop(0, (N_CHUNKS - 1) // 2, body, 0)

        wait_gather(N_CHUNKS - 1, db0)
        scatter_add(N_CHUNKS - 1, db0)

        with jax.named_scope("copy_out"):
            plsc.subcore_barrier()
            col = pl.multiple_of(c * HALF, HALF)
            pltpu.sync_copy(
                acc.at[pl.ds(s * OUT_TILE, OUT_TILE)],
                out_hbm.at[pl.ds(s * OUT_TILE, OUT_TILE), pl.ds(col, HALF)])

            @pl.when(s == N_TILES - 1)
            def _():
                base = N_TILES * OUT_TILE
                pltpu.sync_copy(
                    acc.at[pl.ds(base, N_NODES - base)],
                    out_hbm.at[pl.ds(base, N_NODES - base), pl.ds(col, HALF)])

    return agg(h2, src2, dst_r, zeros)


def kernel(x, edge_index, W):
    src = edge_index[0].astype(jnp.int32)
    dst = edge_index[1].astype(jnp.int32)

    h2 = _matmul_halves(x, W)

    srcp = jnp.pad(src.reshape(N_TILES, E_TILE), ((0, 0), (0, E_PAD)),
                   constant_values=0)
    dstp = jnp.pad(dst.reshape(N_TILES, E_TILE), ((0, 0), (0, E_PAD)),
                   constant_values=DUMMY_ROW)
    src2 = jnp.stack([srcp, srcp + N_NODES]).reshape(-1)
    dst_r = dstp.reshape(N_TILES, N_CHUNKS, K)
    zeros = jnp.zeros((ACC_ROWS, HALF), jnp.float32)

    return _sc_aggregate(h2, src2, dst_r, zeros)


# speedup vs baseline: 1.7448x; 1.0249x over previous
import functools

import jax
import jax.numpy as jnp
from jax import lax
from jax.experimental import pallas as pl
from jax.experimental.pallas import tpu as pltpu
from jax.experimental.pallas import tpu_sc as plsc

N_NODES = 10000
D_IN = 256
D_OUT = 256
N_EDGES = 160000

HALF = D_OUT // 2
N_TILES = 16
E_TILE = N_EDGES // N_TILES
K = 96
N_CHUNKS = 105
E_TILE_P = N_CHUNKS * K
E_PAD = E_TILE_P - E_TILE
ACC_ROWS = 10112
DUMMY_ROW = N_NODES + 16
ROWS_TILE = ACC_ROWS // N_TILES
OUT_TILE = 624


def _mm_body(x_ref, w_ref, o_ref):
    o_ref[...] = jnp.dot(x_ref[...], w_ref[...],
                         preferred_element_type=jnp.float32)


def _matmul_halves(x, W):
    BN = 1000
    return pl.pallas_call(
        _mm_body,
        grid=(2, N_NODES // BN),
        in_specs=[
            pl.BlockSpec((BN, D_IN), lambda c, i: (i, 0)),
            pl.BlockSpec((D_IN, HALF), lambda c, i: (0, c)),
        ],
        out_specs=pl.BlockSpec((BN, HALF), lambda c, i: (c * (N_NODES // BN) + i, 0)),
        out_shape=jax.ShapeDtypeStruct((2 * N_NODES, HALF), jnp.float32),
    )(x, W)


def _sc_aggregate(h2, src2, dst_r):
    mesh = plsc.VectorSubcoreMesh(core_axis_name="c", subcore_axis_name="s")

    @functools.partial(
        pl.kernel,
        mesh=mesh,
        out_type=jax.ShapeDtypeStruct((N_NODES, D_OUT), jnp.float32),
        scratch_types=[
            pltpu.VMEM((E_TILE_P,), jnp.int32),
            pltpu.VMEM((N_CHUNKS, K), jnp.int32),
            pltpu.VMEM((K, HALF), jnp.float32),
            pltpu.VMEM((K, HALF), jnp.float32),
            pltpu.VMEM_SHARED((ACC_ROWS, HALF), jnp.float32),
            pltpu.SemaphoreType.DMA,
        ],
    )
    def agg(h2_hbm, src2_hbm, dst_hbm, out_hbm,
            src_v, dst_v, db0, db1, acc, gsem):
        c = lax.axis_index("c")
        s = lax.axis_index("s")

        with jax.named_scope("stage"):
            z16 = jnp.zeros((16,), jnp.float32)

            def zrow(r, carry):
                for q in range(HALF // 16):
                    db0[r, pl.ds(q * 16, 16)] = z16
                return carry

            lax.fori_loop(0, K, zrow, 0)
            for k in range(ROWS_TILE // K):
                pltpu.sync_copy(db0,
                                acc.at[pl.ds(s * ROWS_TILE + k * K, K)])
            rem = ROWS_TILE % K
            pltpu.sync_copy(
                db0.at[pl.ds(0, rem)],
                acc.at[pl.ds(s * ROWS_TILE + ROWS_TILE - rem, rem)])
            pltpu.sync_copy(
                src2_hbm.at[pl.ds((c * N_TILES + s) * E_TILE_P, E_TILE_P)],
                src_v)
            pltpu.sync_copy(dst_hbm.at[s], dst_v)
            plsc.subcore_barrier()

        def start_gather(j, db):
            pltpu.async_copy(h2_hbm.at[src_v.at[pl.ds(j * K, K)]], db, gsem)

        def wait_gather(j, db):
            pltpu.make_async_copy(h2_hbm.at[src_v.at[pl.ds(j * K, K)]], db,
                                  gsem).wait()

        def scatter_add(j, db):
            pltpu.sync_copy(db, acc.at[dst_v.at[j]], add=True)

        start_gather(0, db0)

        def body(i, carry):
            j = 2 * i
            start_gather(j + 1, db1)
            wait_gather(j, db0)
            scatter_add(j, db0)
            start_gather(j + 2, db0)
            wait_gather(j + 1, db1)
            scatter_add(j + 1, db1)
            return carry

        lax.fori_loop(0, (N_CHUNKS - 1) // 2, body, 0)

        wait_gather(N_CHUNKS - 1, db0)
        scatter_add(N_CHUNKS - 1, db0)

        with jax.named_scope("copy_out"):
            plsc.subcore_barrier()
            col = pl.multiple_of(c * HALF, HALF)
            pltpu.sync_copy(
                acc.at[pl.ds(s * OUT_TILE, OUT_TILE)],
                out_hbm.at[pl.ds(s * OUT_TILE, OUT_TILE), pl.ds(col, HALF)])

            @pl.when(s == N_TILES - 1)
            def _():
                base = N_TILES * OUT_TILE
                pltpu.sync_copy(
                    acc.at[pl.ds(base, N_NODES - base)],
                    out_hbm.at[pl.ds(base, N_NODES - base), pl.ds(col, HALF)])

    return agg(h2, src2, dst_r)


def kernel(x, edge_index, W):
    src = edge_index[0].astype(jnp.int32)
    dst = edge_index[1].astype(jnp.int32)

    h2 = _matmul_halves(x, W)

    srcp = jnp.pad(src.reshape(N_TILES, E_TILE), ((0, 0), (0, E_PAD)),
                   constant_values=0)
    dstp = jnp.pad(dst.reshape(N_TILES, E_TILE), ((0, 0), (0, E_PAD)),
                   constant_values=DUMMY_ROW)
    src2 = jnp.stack([srcp, srcp + N_NODES]).reshape(-1)
    dst_r = dstp.reshape(N_TILES, N_CHUNKS, K)

    return _sc_aggregate(h2, src2, dst_r)
